# Initial kernel scaffold; baseline (speedup 1.0000x reference)
import functools

import jax
import jax.numpy as jnp
from jax import lax
from jax.experimental import pallas as pl
from jax.experimental.pallas import tpu as pltpu
from jax.experimental.pallas import tpu_sc as plsc

B = 4096
T = 200
D = 64
N = B * T

NUM_CORES = 2
NUM_SUBCORES = 16
NW = NUM_CORES * NUM_SUBCORES
PER_W = N // NW
C = 128
CHUNKS = PER_W // C


def _body(pw_hbm, pd_hbm, dow_hbm, tod_hbm, out_hbm,
          idxw_v, idxd_v, rows_w, rows_d, sem_w, sem_d):
  cid = lax.axis_index("c")
  sid = lax.axis_index("s")
  wid = sid * NUM_CORES + cid
  base0 = wid * PER_W

  def chunk(i, carry):
    base = base0 + i * C
    pltpu.sync_copy(pw_hbm.at[pl.ds(base, C)], idxw_v)
    pltpu.sync_copy(pd_hbm.at[pl.ds(base, C)], idxd_v)
    cp_w = pltpu.async_copy(dow_hbm.at[idxw_v], rows_w, sem_w)
    cp_d = pltpu.async_copy(tod_hbm.at[idxd_v], rows_d, sem_d)
    cp_w.wait()
    cp_d.wait()

    def addrow(r, c2):
      for j in range(D // 16):
        sl = pl.ds(j * 16, 16)
        rows_w[r, sl] = rows_w[r, sl] + rows_d[r, sl]
      return c2

    lax.fori_loop(0, C, addrow, None)
    pltpu.sync_copy(rows_w, out_hbm.at[pl.ds(base, C)])
    return carry

  lax.fori_loop(0, CHUNKS, chunk, None)


@jax.jit
def _run(pw, pd, dow_table, tod_table):
  mesh = plsc.VectorSubcoreMesh(core_axis_name="c", subcore_axis_name="s")
  k = pl.kernel(
      _body,
      out_type=jax.ShapeDtypeStruct((N, D), jnp.float32),
      mesh=mesh,
      scratch_types=[
          pltpu.VMEM((C,), jnp.int32),
          pltpu.VMEM((C,), jnp.int32),
          pltpu.VMEM((C, D), jnp.float32),
          pltpu.VMEM((C, D), jnp.float32),
          pltpu.SemaphoreType.DMA,
          pltpu.SemaphoreType.DMA,
      ],
  )
  return k(pw, pd, dow_table, tod_table)


def kernel(pos_w, pos_d, dow_table, tod_table):
  pw = pos_w.reshape(N).astype(jnp.int32)
  pd = pos_d.reshape(N).astype(jnp.int32)
  out = _run(pw, pd, dow_table, tod_table)
  return out.reshape(B, T, 1, D)


# speedup vs baseline: 1.0516x; 1.0516x over previous
import functools

import jax
import jax.numpy as jnp
from jax import lax
from jax.experimental import pallas as pl
from jax.experimental.pallas import tpu as pltpu
from jax.experimental.pallas import tpu_sc as plsc

B = 4096
T = 200
D = 64
N = B * T

NUM_CORES = 2
NUM_SUBCORES = 16
NW = NUM_CORES * NUM_SUBCORES
PER_W = N // NW
C = 128
CHUNKS = PER_W // C


def _body(pw_hbm, pd_hbm, dow_hbm, tod_hbm, out_hbm,
          idxw_v, idxd_v, rows_w, rows_d, sem_w, sem_d):
  cid = lax.axis_index("c")
  sid = lax.axis_index("s")
  wid = sid * NUM_CORES + cid
  base0 = wid * PER_W

  def chunk(i, carry):
    base = base0 + i * C
    pltpu.sync_copy(pw_hbm.at[pl.ds(base, C)], idxw_v)
    pltpu.sync_copy(pd_hbm.at[pl.ds(base, C)], idxd_v)
    cp_w = pltpu.async_copy(dow_hbm.at[idxw_v], rows_w, sem_w)
    cp_d = pltpu.async_copy(tod_hbm.at[idxd_v], rows_d, sem_d)
    cp_w.wait()
    cp_d.wait()

    def addrow(r, c2):
      for j in range(D // 16):
        sl = pl.ds(j * 16, 16)
        rows_w[r, sl] = rows_w[r, sl] + rows_d[r, sl]
      return c2

    lax.fori_loop(0, C, addrow, None)
    pltpu.sync_copy(rows_w, out_hbm.at[pl.ds(base, C)])
    return carry

  lax.fori_loop(0, CHUNKS, chunk, None)


@jax.jit
def _run(pw, pd, dow_table, tod_table):
  mesh = plsc.VectorSubcoreMesh(core_axis_name="c", subcore_axis_name="s")
  k = pl.kernel(
      _body,
      out_type=jax.ShapeDtypeStruct((N, D), jnp.float32),
      mesh=mesh,
      scratch_types=[
          pltpu.VMEM((C,), jnp.int32),
          pltpu.VMEM((C,), jnp.int32),
          pltpu.VMEM((C, D), jnp.float32),
          pltpu.VMEM((C, D), jnp.float32),
          pltpu.SemaphoreType.DMA,
          pltpu.SemaphoreType.DMA,
      ],
      compiler_params=pltpu.CompilerParams(use_tc_tiling_on_sc=False),
  )
  return k(pw, pd, dow_table, tod_table)


def kernel(pos_w, pos_d, dow_table, tod_table):
  pw = pos_w.reshape(N).astype(jnp.int32)
  pd = pos_d.reshape(N).astype(jnp.int32)
  out = _run(pw, pd, dow_table, tod_table)
  return out.reshape(B, T, 1, D)


# speedup vs baseline: 1.0536x; 1.0018x over previous
import jax
import jax.numpy as jnp
from jax import lax
from jax.experimental import pallas as pl
from jax.experimental.pallas import tpu as pltpu
from jax.experimental.pallas import tpu_sc as plsc

B = 4096
T = 200
D = 64
N = B * T

NUM_CORES = 2
NUM_SUBCORES = 16
NW = NUM_CORES * NUM_SUBCORES
PER_W = N // NW
G = 128
C = 256
SUB = C // G
CHUNKS = PER_W // C


def _body(pw_hbm, pd_hbm, dow_hbm, tod_hbm, out_hbm,
          iw0, iw1, id0, id1, rw0, rw1, rd0, rd1,
          siw0, siw1, sid0, sid1, sw0, sw1, sd0, sd1, so0, so1):
  iw = (iw0, iw1)
  idx_d = (id0, id1)
  rw = (rw0, rw1)
  rd = (rd0, rd1)
  siw = (siw0, siw1)
  sid = (sid0, sid1)
  sw = (sw0, sw1)
  sd = (sd0, sd1)
  so = (so0, so1)

  cid = lax.axis_index("c")
  sid_ = lax.axis_index("s")
  wid = sid_ * NUM_CORES + cid
  base0 = wid * PER_W

  def idx_start(i, s):
    base = base0 + i * C
    for j in range(SUB):
      pltpu.async_copy(pw_hbm.at[pl.ds(base + j * G, G)], iw[s].at[j], siw[s])
      pltpu.async_copy(pd_hbm.at[pl.ds(base + j * G, G)], idx_d[s].at[j], sid[s])

  def idx_wait(s):
    for j in range(SUB):
      pltpu.make_async_copy(pw_hbm.at[pl.ds(0, G)], iw[s].at[j], siw[s]).wait()
      pltpu.make_async_copy(pd_hbm.at[pl.ds(0, G)], idx_d[s].at[j], sid[s]).wait()

  def gather_start(s):
    for j in range(SUB):
      sl = pl.ds(j * G, G)
      pltpu.async_copy(dow_hbm.at[iw[s].at[j]], rw[s].at[sl], sw[s])
      pltpu.async_copy(tod_hbm.at[idx_d[s].at[j]], rd[s].at[sl], sd[s])

  def gather_wait(s):
    for j in range(SUB):
      sl = pl.ds(j * G, G)
      pltpu.make_async_copy(dow_hbm.at[iw[s].at[j]], rw[s].at[sl], sw[s]).wait()
      pltpu.make_async_copy(tod_hbm.at[idx_d[s].at[j]], rd[s].at[sl], sd[s]).wait()

  def out_start(i, s):
    base = base0 + i * C
    pltpu.async_copy(rw[s], out_hbm.at[pl.ds(base, C)], so[s])

  def out_wait(s):
    pltpu.make_async_copy(rw[s], out_hbm.at[pl.ds(0, C)], so[s]).wait()

  def compute(s):
    @plsc.parallel_loop(0, C, 1, unroll=4)
    def _(r):
      for j in range(D // 16):
        sl = pl.ds(j * 16, 16)
        plsc.addupdate(rw[s].at[r, sl], rd[s][r, sl])

  idx_start(0, 0)
  idx_start(1, 1)
  idx_wait(0)
  gather_start(0)

  def step(i, s):
    gather_wait(s)

    @pl.when(i + 1 < CHUNKS)
    def _():
      @pl.when(i >= 1)
      def _():
        out_wait(1 - s)
      idx_wait(1 - s)
      gather_start(1 - s)

    @pl.when(i + 2 < CHUNKS)
    def _():
      idx_start(i + 2, s)

    compute(s)
    out_start(i, s)

  def group(g, carry):
    step(2 * g, 0)
    step(2 * g + 1, 1)
    return carry

  lax.fori_loop(0, CHUNKS // 2, group, None)
  out_wait(0)
  out_wait(1)


@jax.jit
def _run(pw, pd, dow_table, tod_table):
  mesh = plsc.VectorSubcoreMesh(core_axis_name="c", subcore_axis_name="s")
  k = pl.kernel(
      _body,
      out_type=jax.ShapeDtypeStruct((N, D), jnp.float32),
      mesh=mesh,
      scratch_types=[
          pltpu.VMEM((SUB, G), jnp.int32),
          pltpu.VMEM((SUB, G), jnp.int32),
          pltpu.VMEM((SUB, G), jnp.int32),
          pltpu.VMEM((SUB, G), jnp.int32),
          pltpu.VMEM((C, D), jnp.float32),
          pltpu.VMEM((C, D), jnp.float32),
          pltpu.VMEM((C, D), jnp.float32),
          pltpu.VMEM((C, D), jnp.float32),
          pltpu.SemaphoreType.DMA,
          pltpu.SemaphoreType.DMA,
          pltpu.SemaphoreType.DMA,
          pltpu.SemaphoreType.DMA,
          pltpu.SemaphoreType.DMA,
          pltpu.SemaphoreType.DMA,
          pltpu.SemaphoreType.DMA,
          pltpu.SemaphoreType.DMA,
          pltpu.SemaphoreType.DMA,
          pltpu.SemaphoreType.DMA,
      ],
      compiler_params=pltpu.CompilerParams(use_tc_tiling_on_sc=False),
  )
  return k(pw, pd, dow_table, tod_table)


def kernel(pos_w, pos_d, dow_table, tod_table):
  pw = pos_w.reshape(N).astype(jnp.int32)
  pd = pos_d.reshape(N).astype(jnp.int32)
  out = _run(pw, pd, dow_table, tod_table)
  return out.reshape(B, T, 1, D)


# speedup vs baseline: 8.3829x; 7.9568x over previous
import jax
import jax.numpy as jnp
from jax import lax
from jax.experimental import pallas as pl
from jax.experimental.pallas import tpu as pltpu
from jax.experimental.pallas import tpu_sc as plsc

B = 4096
T = 200
D = 64
N = B * T
WEEK = 7
DAY = 288

NUM_CORES = 2
NUM_SUBCORES = 16
NW = NUM_CORES * NUM_SUBCORES
PER_W = N // NW
C = 512
CHUNKS = PER_W // C


def _body(pw_hbm, pd_hbm, dow_hbm, tod_hbm, out_hbm,
          dow_l, tod_l, iw0, iw1, id0, id1, ob0, ob1,
          si0, si1, sob0, sob1, stab):
  iw = (iw0, iw1)
  idd = (id0, id1)
  ob = (ob0, ob1)
  si = (si0, si1)
  sob = (sob0, sob1)

  cid = lax.axis_index("c")
  sid_ = lax.axis_index("s")
  wid = sid_ * NUM_CORES + cid
  base0 = wid * PER_W

  cp1 = pltpu.async_copy(dow_hbm, dow_l, stab)
  cp2 = pltpu.async_copy(tod_hbm, tod_l, stab)

  def idx_start(i, s):
    base = base0 + i * C
    pltpu.async_copy(pw_hbm.at[pl.ds(base, C)], iw[s], si[s])
    pltpu.async_copy(pd_hbm.at[pl.ds(base, C)], idd[s], si[s])

  def idx_wait(s):
    pltpu.make_async_copy(pw_hbm.at[pl.ds(0, C)], iw[s], si[s]).wait()
    pltpu.make_async_copy(pd_hbm.at[pl.ds(0, C)], idd[s], si[s]).wait()

  def out_start(i, s):
    base = base0 + i * C
    pltpu.async_copy(ob[s], out_hbm.at[pl.ds(base, C)], sob[s])

  def out_wait(s):
    pltpu.make_async_copy(ob[s], out_hbm.at[pl.ds(0, C)], sob[s]).wait()

  def compute(s):
    @plsc.parallel_loop(0, C // 16, 1)
    def _(g):
      wv = iw[s][pl.ds(g * 16, 16)]
      dv = idd[s][pl.ds(g * 16, 16)]
      for jj in range(16):
        w = wv[jj]
        d = dv[jj]
        r = g * 16 + jj
        for j in range(D // 16):
          sl = pl.ds(j * 16, 16)
          ob[s][r, sl] = dow_l[w, sl] + tod_l[d, sl]

  idx_start(0, 0)
  idx_start(1, 1)
  cp1.wait()
  cp2.wait()

  def step(i, s):
    idx_wait(s)

    @pl.when(i >= 2)
    def _():
      out_wait(s)

    compute(s)
    out_start(i, s)

    @pl.when(i + 2 < CHUNKS)
    def _():
      idx_start(i + 2, s)

  def group(g, carry):
    step(2 * g, 0)
    step(2 * g + 1, 1)
    return carry

  lax.fori_loop(0, CHUNKS // 2, group, None)
  out_wait(0)
  out_wait(1)


@jax.jit
def _run(pw, pd, dow_table, tod_table):
  mesh = plsc.VectorSubcoreMesh(core_axis_name="c", subcore_axis_name="s")
  k = pl.kernel(
      _body,
      out_type=jax.ShapeDtypeStruct((N, D), jnp.float32),
      mesh=mesh,
      scratch_types=[
          pltpu.VMEM((WEEK, D), jnp.float32),
          pltpu.VMEM((DAY, D), jnp.float32),
          pltpu.VMEM((C,), jnp.int32),
          pltpu.VMEM((C,), jnp.int32),
          pltpu.VMEM((C,), jnp.int32),
          pltpu.VMEM((C,), jnp.int32),
          pltpu.VMEM((C, D), jnp.float32),
          pltpu.VMEM((C, D), jnp.float32),
          pltpu.SemaphoreType.DMA,
          pltpu.SemaphoreType.DMA,
          pltpu.SemaphoreType.DMA,
          pltpu.SemaphoreType.DMA,
          pltpu.SemaphoreType.DMA,
      ],
      compiler_params=pltpu.CompilerParams(use_tc_tiling_on_sc=False),
  )
  return k(pw, pd, dow_table, tod_table)


def kernel(pos_w, pos_d, dow_table, tod_table):
  pw = pos_w.reshape(N).astype(jnp.int32)
  pd = pos_d.reshape(N).astype(jnp.int32)
  out = _run(pw, pd, dow_table, tod_table)
  return out.reshape(B, T, 1, D)
